# baseline (speedup 1.0000x reference)
import jax
import jax.numpy as jnp
from jax.experimental import pallas as pl
from jax.experimental.pallas import tpu as pltpu

_BM = 512
_BK = 512
_NS = 32


def _vq_kernel(zsqh_ref, csqh_ref, z_ref, tab_ref, out_ref, rmin_ref, rpk_ref):
    j = pl.program_id(0)
    i = pl.program_id(1)
    nj = pl.num_programs(0)
    np_ = _BK // _NS

    tb = tab_ref[...]
    csqh = csqh_ref[...]
    zs = z_ref[pl.ds(i * _BM, _BM), :]
    zsqh = zsqh_ref[...].reshape(1, _BM)

    cross = jax.lax.dot_general(
        tb, zs, (((1,), (1,)), ((), ())),
        preferred_element_type=jnp.float32)
    dt = zsqh - cross + csqh
    d3 = dt.reshape(np_, _NS, _BM)

    srow = pl.ds(i * _NS, _NS)
    rmin = jnp.where(j == 0, jnp.inf, rmin_ref[srow, :])
    rpk = jnp.where(j == 0, 0, rpk_ref[srow, :])
    base = j * np_
    for p in range(np_):
        dq = d3[p]
        upd = dq < rmin
        rmin = jnp.where(upd, dq, rmin)
        rpk = jnp.where(upd, base + p, rpk)
    rmin_ref[srow, :] = rmin
    rpk_ref[srow, :] = rpk

    @pl.when(j == nj - 1)
    def _finish():
        sio = jax.lax.broadcasted_iota(jnp.int32, (_NS, _BM), 0)
        v, k = rmin, rpk * _NS + sio
        s = _NS
        while s > 1:
            sh = s // 2
            va, vb = v[:sh, :], v[sh:s, :]
            ka, kb = k[:sh, :], k[sh:s, :]
            take_b = (vb < va) | ((vb == va) & (kb < ka))
            v = jnp.where(take_b, vb, va)
            k = jnp.where(take_b, kb, ka)
            s = sh
        out_ref[...] = k.reshape(_BM)


def kernel(z_e_x, table):
    B, D = z_e_x.shape
    K, _ = table.shape
    nb = B // _BM
    z_sq_h = (0.5 * jnp.sum(z_e_x * z_e_x, axis=-1)).reshape(nb, 1, _BM)
    c_sq_h = (0.5 * jnp.sum(table * table, axis=-1))[:, None]
    return pl.pallas_call(
        _vq_kernel,
        grid=(K // _BK, nb),
        in_specs=[
            pl.BlockSpec((1, 1, _BM), lambda j, i: (i, 0, 0)),
            pl.BlockSpec((_BK, 1), lambda j, i: (j, 0)),
            pl.BlockSpec((B, D), lambda j, i: (0, 0)),
            pl.BlockSpec((_BK, D), lambda j, i: (j, 0)),
        ],
        out_specs=pl.BlockSpec((_BM,), lambda j, i: (i,)),
        out_shape=jax.ShapeDtypeStruct((B,), jnp.int32),
        scratch_shapes=[
            pltpu.VMEM((nb * _NS, _BM), jnp.float32),
            pltpu.VMEM((nb * _NS, _BM), jnp.int32),
        ],
    )(z_sq_h, c_sq_h, z_e_x, table)


# speedup vs baseline: 2.3063x; 2.3063x over previous
import jax
import jax.numpy as jnp
from jax.experimental import pallas as pl

_BM = 512
_BK = 512
_NS = 32


def _vq_kernel(zsqh_ref, csqh_ref, z_ref, tab_ref, out_ref):
    z = z_ref[...]
    zsqh = zsqh_ref[...]
    K = tab_ref.shape[0]
    num_k = K // _BK
    np_ = _BK // _NS

    rmin = jnp.full((_NS, _BM), jnp.inf, dtype=jnp.float32)
    rpk = jnp.zeros((_NS, _BM), dtype=jnp.int32)
    for j in range(num_k):
        tb = tab_ref[j * _BK:(j + 1) * _BK, :]
        cross = jax.lax.dot_general(
            tb, z, (((1,), (1,)), ((), ())),
            preferred_element_type=jnp.float32)
        csqh = csqh_ref[j * _BK:(j + 1) * _BK, :]
        dt = zsqh - cross + csqh
        d3 = dt.reshape(np_, _NS, _BM)
        for p in range(np_):
            dq = d3[p]
            upd = dq < rmin
            rmin = jnp.where(upd, dq, rmin)
            rpk = jnp.where(upd, jnp.int32(j * np_ + p), rpk)

    sio = jax.lax.broadcasted_iota(jnp.int32, (_NS, _BM), 0)
    v, k = rmin, rpk * _NS + sio
    s = _NS
    while s > 1:
        sh = s // 2
        va, vb = v[:sh, :], v[sh:s, :]
        ka, kb = k[:sh, :], k[sh:s, :]
        take_b = (vb < va) | ((vb == va) & (kb < ka))
        v = jnp.where(take_b, vb, va)
        k = jnp.where(take_b, kb, ka)
        s = sh
    out_ref[...] = k.reshape(_BM)


def kernel(z_e_x, table):
    B, D = z_e_x.shape
    K, _ = table.shape
    z_sq_h = (0.5 * jnp.sum(z_e_x * z_e_x, axis=-1))[None, :]
    c_sq_h = (0.5 * jnp.sum(table * table, axis=-1))[:, None]
    return pl.pallas_call(
        _vq_kernel,
        grid=(B // _BM,),
        in_specs=[
            pl.BlockSpec((1, _BM), lambda i: (0, i)),
            pl.BlockSpec((K, 1), lambda i: (0, 0)),
            pl.BlockSpec((_BM, D), lambda i: (i, 0)),
            pl.BlockSpec((K, D), lambda i: (0, 0)),
        ],
        out_specs=pl.BlockSpec((_BM,), lambda i: (i,)),
        out_shape=jax.ShapeDtypeStruct((B,), jnp.int32),
    )(z_sq_h, c_sq_h, z_e_x, table)
